# Initial kernel scaffold; baseline (speedup 1.0000x reference)
import jax
import jax.numpy as jnp
from jax.experimental import pallas as pl


def kernel(h, edge_index, coord, vel, edge_attr, We1, be1, We2, be2, Wn1, bn1, Wn2, bn2, Wc1, bc1, Wc2, bc2):
    raise NotImplementedError("write your pallas kernel here")


# speedup vs baseline: 2.5435x; 2.5435x over previous
import functools
import jax
import jax.numpy as jnp
from jax import lax
from jax.experimental import pallas as pl
from jax.experimental.pallas import tpu as pltpu
from jax.experimental.pallas import tpu_sc as plsc

_N = 10000
_E = 320000
_D = 128
_H = 128
_DE = 16
_STEP = 1.0 / 4.0
_TW = 144
_NC = 2
_NS = 16
_NW = _NC * _NS
_EPW = _E // _NW
_CH = 80
_NCHUNK = _EPW // _CH
_BN = 512
_BE = 512
_NROWS_PER_SUB = _N // _NS


def _silu(x):
    return x * jax.nn.sigmoid(x)


def _prep_body(h_ref, cp_ref, WaT_ref, WbT_ref, be1_ref, ta_ref, tb_ref):
    h = h_ref[...]
    ta_ref[:, :_D] = jnp.dot(h, WaT_ref[...], preferred_element_type=jnp.float32) + be1_ref[...]
    ta_ref[:, _D:] = cp_ref[...]
    tb_ref[:, :_D] = jnp.dot(h, WbT_ref[...], preferred_element_type=jnp.float32)
    tb_ref[:, _D:] = cp_ref[...]


def _prep_call(h, coordp, WaT, WbT, be1r):
    nblk = pl.cdiv(_N, _BN)
    return pl.pallas_call(
        _prep_body,
        grid=(nblk,),
        in_specs=[
            pl.BlockSpec((_BN, _D), lambda i: (i, 0)),
            pl.BlockSpec((_BN, _TW - _D), lambda i: (i, 0)),
            pl.BlockSpec((_D, _D), lambda i: (0, 0)),
            pl.BlockSpec((_D, _D), lambda i: (0, 0)),
            pl.BlockSpec((1, _D), lambda i: (0, 0)),
        ],
        out_specs=[
            pl.BlockSpec((_BN, _TW), lambda i: (i, 0)),
            pl.BlockSpec((_BN, _TW), lambda i: (i, 0)),
        ],
        out_shape=[
            jax.ShapeDtypeStruct((_N, _TW), jnp.float32),
            jax.ShapeDtypeStruct((_N, _TW), jnp.float32),
        ],
    )(h, coordp, WaT, WbT, be1r)


def _gather_body(row_hbm, col_hbm, ta_hbm, tb_hbm, ga_hbm, gb_hbm,
                 idxr_v, idxc_v, bufa_v, bufb_v, sema, semb):
    cid = lax.axis_index("c")
    sid = lax.axis_index("s")
    wid = cid * _NS + sid
    base = wid * _EPW

    pltpu.sync_copy(row_hbm.at[pl.ds(base, _EPW)], idxr_v)
    pltpu.sync_copy(col_hbm.at[pl.ds(base, _EPW)], idxc_v)

    def start(k, b):
        o = k * _CH
        pltpu.async_copy(ta_hbm.at[idxr_v.at[pl.ds(o, _CH)]], bufa_v.at[b],
                         sema.at[b])
        pltpu.async_copy(tb_hbm.at[idxc_v.at[pl.ds(o, _CH)]], bufb_v.at[b],
                         semb.at[b])

    def finish(k, b):
        o = k * _CH
        pltpu.make_async_copy(ta_hbm.at[idxr_v.at[pl.ds(o, _CH)]],
                              bufa_v.at[b], sema.at[b]).wait()
        pltpu.make_async_copy(tb_hbm.at[idxc_v.at[pl.ds(o, _CH)]],
                              bufb_v.at[b], semb.at[b]).wait()
        pltpu.sync_copy(bufa_v.at[b], ga_hbm.at[pl.ds(base + o, _CH)])
        pltpu.sync_copy(bufb_v.at[b], gb_hbm.at[pl.ds(base + o, _CH)])

    start(0, 0)

    def body(k, carry):
        b = lax.rem(k, 2)
        @pl.when(k + 1 < _NCHUNK)
        def _():
            start(k + 1, 1 - b)
        finish(k, b)
        return carry

    lax.fori_loop(0, _NCHUNK, body, 0)


def _gather_call(row, col, ta, tb):
    mesh = plsc.VectorSubcoreMesh(core_axis_name="c", subcore_axis_name="s",
                                  num_cores=_NC, num_subcores=_NS)
    f = functools.partial(
        pl.kernel,
        out_type=[
            jax.ShapeDtypeStruct((_E, _TW), jnp.float32),
            jax.ShapeDtypeStruct((_E, _TW), jnp.float32),
        ],
        mesh=mesh,
        compiler_params=pltpu.CompilerParams(use_tc_tiling_on_sc=False),
        scratch_types=[
            pltpu.VMEM((_EPW,), jnp.int32),
            pltpu.VMEM((_EPW,), jnp.int32),
            pltpu.VMEM((2, _CH, _TW), jnp.float32),
            pltpu.VMEM((2, _CH, _TW), jnp.float32),
            pltpu.SemaphoreType.DMA((2,)),
            pltpu.SemaphoreType.DMA((2,)),
        ],
    )(_gather_body)
    return f(row, col, ta, tb)


def _edge_body(ga_ref, gb_ref, ea_ref, WeAT_ref, wr_ref, We2T_ref, be2_ref,
               Wc1T_ref, bc1_ref, wc2_ref, bc2_ref, efx_ref):
    ga = ga_ref[...]
    gb = gb_ref[...]
    cd = ga[:, _D:] - gb[:, _D:]
    radial = jnp.sum(cd * cd, axis=1, keepdims=True)
    t1 = (ga[:, :_D] + gb[:, :_D] + radial * wr_ref[...]
          + jnp.dot(ea_ref[...], WeAT_ref[...], preferred_element_type=jnp.float32))
    t1 = _silu(t1)
    ef = jnp.dot(t1, We2T_ref[...], preferred_element_type=jnp.float32) + be2_ref[...]
    ef = _silu(ef)
    cv = jnp.dot(ef, Wc1T_ref[...], preferred_element_type=jnp.float32) + bc1_ref[...]
    cv = _silu(cv)
    c = jnp.dot(cv, wc2_ref[...], preferred_element_type=jnp.float32) + bc2_ref[...]
    trans = jnp.clip(cd * c, -100.0, 100.0)
    lane = lax.broadcasted_iota(jnp.int32, trans.shape, 1)
    aux = jnp.where(lane == 3, 1.0, trans)
    efx_ref[:, :_D] = ef
    efx_ref[:, _D:] = aux


def _edge_call(ga, gb, edge_attr, WeAT, wrr, We2T, be2r, Wc1T, bc1r, wc2, bc2r):
    nblk = _E // _BE
    return pl.pallas_call(
        _edge_body,
        grid=(nblk,),
        in_specs=[
            pl.BlockSpec((_BE, _TW), lambda i: (i, 0)),
            pl.BlockSpec((_BE, _TW), lambda i: (i, 0)),
            pl.BlockSpec((_BE, _DE), lambda i: (i, 0)),
            pl.BlockSpec((_DE, _H), lambda i: (0, 0)),
            pl.BlockSpec((1, _H), lambda i: (0, 0)),
            pl.BlockSpec((_H, _H), lambda i: (0, 0)),
            pl.BlockSpec((1, _H), lambda i: (0, 0)),
            pl.BlockSpec((_H, _H), lambda i: (0, 0)),
            pl.BlockSpec((1, _H), lambda i: (0, 0)),
            pl.BlockSpec((_H, 1), lambda i: (0, 0)),
            pl.BlockSpec((1, 1), lambda i: (0, 0)),
        ],
        out_specs=pl.BlockSpec((_BE, _TW), lambda i: (i, 0)),
        out_shape=jax.ShapeDtypeStruct((_E, _TW), jnp.float32),
    )(ga, gb, edge_attr, WeAT, wrr, We2T, be2r, Wc1T, bc1r, wc2, bc2r)


def _scatter_body(efx_hbm, row_hbm, zeros_hbm, out_hbm, idx_v, data_v, acc_sh,
                  semi, semd):
    cid = lax.axis_index("c")
    sid = lax.axis_index("s")
    wid = cid * _NS + sid
    base = wid * _EPW

    rbase = sid * _NROWS_PER_SUB
    pltpu.sync_copy(zeros_hbm.at[pl.ds(rbase, _NROWS_PER_SUB)],
                    acc_sh.at[pl.ds(rbase, _NROWS_PER_SUB)])
    plsc.subcore_barrier()

    def start(k, b):
        off = base + k * _CH
        pltpu.async_copy(row_hbm.at[pl.ds(off, _CH)], idx_v.at[b],
                         semi.at[b])
        pltpu.async_copy(efx_hbm.at[pl.ds(off, _CH)], data_v.at[b],
                         semd.at[b])

    def finish(k, b):
        off = base + k * _CH
        pltpu.make_async_copy(row_hbm.at[pl.ds(off, _CH)], idx_v.at[b],
                              semi.at[b]).wait()
        pltpu.make_async_copy(efx_hbm.at[pl.ds(off, _CH)], data_v.at[b],
                              semd.at[b]).wait()
        pltpu.sync_copy(data_v.at[b], acc_sh.at[idx_v.at[b]], add=True)

    start(0, 0)

    def body(k, carry):
        b = lax.rem(k, 2)
        @pl.when(k + 1 < _NCHUNK)
        def _():
            start(k + 1, 1 - b)
        finish(k, b)
        return carry

    lax.fori_loop(0, _NCHUNK, body, 0)

    plsc.subcore_barrier()
    pltpu.sync_copy(acc_sh.at[pl.ds(rbase, _NROWS_PER_SUB)],
                    out_hbm.at[cid, pl.ds(rbase, _NROWS_PER_SUB)])


def _scatter_call(efx, row, zeros_nt):
    mesh = plsc.VectorSubcoreMesh(core_axis_name="c", subcore_axis_name="s",
                                  num_cores=_NC, num_subcores=_NS)
    f = functools.partial(
        pl.kernel,
        out_type=jax.ShapeDtypeStruct((_NC, _N, _TW), jnp.float32),
        mesh=mesh,
        compiler_params=pltpu.CompilerParams(use_tc_tiling_on_sc=False),
        scratch_types=[
            pltpu.VMEM((2, _CH), jnp.int32),
            pltpu.VMEM((2, _CH, _TW), jnp.float32),
            pltpu.VMEM_SHARED((_N, _TW), jnp.float32),
            pltpu.SemaphoreType.DMA((2,)),
            pltpu.SemaphoreType.DMA((2,)),
        ],
    )(_scatter_body)
    return f(efx, row, zeros_nt)


def _node_body(h_ref, acc_ref, coord_ref, vel_ref, Wn1hT_ref, Wn1aT_ref,
               bn1_ref, Wn2T_ref, bn2_ref, h2_ref, coord2_ref, vel2_ref):
    acc = acc_ref[0] + acc_ref[1]
    agg = acc[:, :_D]
    trans_sum = acc[:, _D:_D + 3]
    cnt = jnp.clip(acc[:, _D + 3:_D + 4], 1.0, None)
    a_like = trans_sum / cnt
    vel2 = vel_ref[...] + a_like * _STEP
    vel2_ref[...] = vel2
    coord2_ref[...] = coord_ref[...] + vel2 * _STEP
    h = h_ref[...]
    n1 = (jnp.dot(h, Wn1hT_ref[...], preferred_element_type=jnp.float32)
          + jnp.dot(agg, Wn1aT_ref[...], preferred_element_type=jnp.float32)
          + bn1_ref[...])
    n1 = _silu(n1)
    h2_ref[...] = h + jnp.dot(n1, Wn2T_ref[...], preferred_element_type=jnp.float32) + bn2_ref[...]


def _node_call(h, acc, coord, vel, Wn1hT, Wn1aT, bn1r, Wn2T, bn2r):
    nblk = pl.cdiv(_N, _BN)
    return pl.pallas_call(
        _node_body,
        grid=(nblk,),
        in_specs=[
            pl.BlockSpec((_BN, _D), lambda i: (i, 0)),
            pl.BlockSpec((_NC, _BN, _TW), lambda i: (0, i, 0)),
            pl.BlockSpec((_BN, 3), lambda i: (i, 0)),
            pl.BlockSpec((_BN, 3), lambda i: (i, 0)),
            pl.BlockSpec((_D, _H), lambda i: (0, 0)),
            pl.BlockSpec((_H, _H), lambda i: (0, 0)),
            pl.BlockSpec((1, _H), lambda i: (0, 0)),
            pl.BlockSpec((_H, _D), lambda i: (0, 0)),
            pl.BlockSpec((1, _D), lambda i: (0, 0)),
        ],
        out_specs=[
            pl.BlockSpec((_BN, _D), lambda i: (i, 0)),
            pl.BlockSpec((_BN, 3), lambda i: (i, 0)),
            pl.BlockSpec((_BN, 3), lambda i: (i, 0)),
        ],
        out_shape=[
            jax.ShapeDtypeStruct((_N, _D), jnp.float32),
            jax.ShapeDtypeStruct((_N, 3), jnp.float32),
            jax.ShapeDtypeStruct((_N, 3), jnp.float32),
        ],
    )(h, acc, coord, vel, Wn1hT, Wn1aT, bn1r, Wn2T, bn2r)


@jax.jit
def kernel(h, edge_index, coord, vel, edge_attr, We1, be1, We2, be2,
           Wn1, bn1, Wn2, bn2, Wc1, bc1, Wc2, bc2):
    row = edge_index[0]
    col = edge_index[1]
    coordp = jnp.pad(coord, ((0, 0), (0, _TW - _D - 3)))
    WaT = We1[:, :_D].T
    WbT = We1[:, _D:2 * _D].T
    wrr = We1[:, 2 * _D].reshape(1, _H)
    WeAT = We1[:, 2 * _D + 1:].T
    be1r = be1.reshape(1, _H)
    be2r = be2.reshape(1, _H)
    bc1r = bc1.reshape(1, _H)
    wc2 = Wc2.T
    bc2r = bc2.reshape(1, 1)
    Wn1hT = Wn1[:, :_D].T
    Wn1aT = Wn1[:, _D:].T
    bn1r = bn1.reshape(1, _H)
    Wn2T = Wn2.T
    bn2r = bn2.reshape(1, _D)
    zeros_nt = jnp.zeros((_N, _TW), jnp.float32)

    ta, tb = _prep_call(h, coordp, WaT, WbT, be1r)
    ga, gb = _gather_call(row, col, ta, tb)
    efx = _edge_call(ga, gb, edge_attr, WeAT, wrr, We2T := We2.T, be2r,
                     Wc1.T, bc1r, wc2, bc2r)
    acc = _scatter_call(efx, row, zeros_nt)
    h2, coord2, vel2 = _node_call(h, acc, coord, vel, Wn1hT, Wn1aT, bn1r,
                                  Wn2T, bn2r)
    return (h2, coord2, vel2)


# speedup vs baseline: 2.6592x; 1.0455x over previous
import functools
import jax
import jax.numpy as jnp
from jax import lax
from jax.experimental import pallas as pl
from jax.experimental.pallas import tpu as pltpu
from jax.experimental.pallas import tpu_sc as plsc

_N = 10000
_E = 320000
_D = 128
_H = 128
_DE = 16
_STEP = 1.0 / 4.0
_TW = 144
_NC = 2
_NS = 16
_NW = _NC * _NS
_EPW = _E // _NW
_CH = 80
_NCHUNK = _EPW // _CH
_BN = 512
_BE = 640
_NROWS_PER_SUB = _N // _NS


def _silu(x):
    return x * jax.nn.sigmoid(x)


def _prep_body(h_ref, cp_ref, WaT_ref, WbT_ref, be1_ref, ta_ref, tb_ref):
    h = h_ref[...]
    ta_ref[:, :_D] = jnp.dot(h, WaT_ref[...], preferred_element_type=jnp.float32) + be1_ref[...]
    ta_ref[:, _D:] = cp_ref[...]
    tb_ref[:, :_D] = jnp.dot(h, WbT_ref[...], preferred_element_type=jnp.float32)
    tb_ref[:, _D:] = cp_ref[...]


def _prep_call(h, coordp, WaT, WbT, be1r):
    nblk = pl.cdiv(_N, _BN)
    return pl.pallas_call(
        _prep_body,
        grid=(nblk,),
        in_specs=[
            pl.BlockSpec((_BN, _D), lambda i: (i, 0)),
            pl.BlockSpec((_BN, _TW - _D), lambda i: (i, 0)),
            pl.BlockSpec((_D, _D), lambda i: (0, 0)),
            pl.BlockSpec((_D, _D), lambda i: (0, 0)),
            pl.BlockSpec((1, _D), lambda i: (0, 0)),
        ],
        out_specs=[
            pl.BlockSpec((_BN, _TW), lambda i: (i, 0)),
            pl.BlockSpec((_BN, _TW), lambda i: (i, 0)),
        ],
        out_shape=[
            jax.ShapeDtypeStruct((_N, _TW), jnp.float32),
            jax.ShapeDtypeStruct((_N, _TW), jnp.float32),
        ],
    )(h, coordp, WaT, WbT, be1r)


def _gather_body(ei_hbm, ta_hbm, tb_hbm, ga_hbm, gb_hbm,
                 idxr_v, idxc_v, bufa_v, bufb_v, sema, semb):
    cid = lax.axis_index("c")
    sid = lax.axis_index("s")
    wid = cid * _NS + sid
    base = wid * _EPW

    pltpu.sync_copy(ei_hbm.at[0, pl.ds(base, _EPW)], idxr_v)
    pltpu.sync_copy(ei_hbm.at[1, pl.ds(base, _EPW)], idxc_v)

    def start(k, b):
        o = k * _CH
        pltpu.async_copy(ta_hbm.at[idxr_v.at[pl.ds(o, _CH)]], bufa_v.at[b],
                         sema.at[b])
        pltpu.async_copy(tb_hbm.at[idxc_v.at[pl.ds(o, _CH)]], bufb_v.at[b],
                         semb.at[b])

    def finish(k, b):
        o = k * _CH
        pltpu.make_async_copy(ta_hbm.at[idxr_v.at[pl.ds(o, _CH)]],
                              bufa_v.at[b], sema.at[b]).wait()
        pltpu.make_async_copy(tb_hbm.at[idxc_v.at[pl.ds(o, _CH)]],
                              bufb_v.at[b], semb.at[b]).wait()
        pltpu.sync_copy(bufa_v.at[b], ga_hbm.at[pl.ds(base + o, _CH)])
        pltpu.sync_copy(bufb_v.at[b], gb_hbm.at[pl.ds(base + o, _CH)])

    start(0, 0)

    def body(k, carry):
        b = lax.rem(k, 2)
        @pl.when(k + 1 < _NCHUNK)
        def _():
            start(k + 1, 1 - b)
        finish(k, b)
        return carry

    lax.fori_loop(0, _NCHUNK, body, 0)


def _gather_call(ei, ta, tb):
    mesh = plsc.VectorSubcoreMesh(core_axis_name="c", subcore_axis_name="s",
                                  num_cores=_NC, num_subcores=_NS)
    f = functools.partial(
        pl.kernel,
        out_type=[
            jax.ShapeDtypeStruct((_E, _TW), jnp.float32),
            jax.ShapeDtypeStruct((_E, _TW), jnp.float32),
        ],
        mesh=mesh,
        compiler_params=pltpu.CompilerParams(use_tc_tiling_on_sc=False),
        scratch_types=[
            pltpu.VMEM((_EPW,), jnp.int32),
            pltpu.VMEM((_EPW,), jnp.int32),
            pltpu.VMEM((2, _CH, _TW), jnp.float32),
            pltpu.VMEM((2, _CH, _TW), jnp.float32),
            pltpu.SemaphoreType.DMA((2,)),
            pltpu.SemaphoreType.DMA((2,)),
        ],
    )(_gather_body)
    return f(ei, ta, tb)


def _edge_body(ga_ref, gb_ref, ea_ref, WeAT_ref, wr_ref, We2T_ref, be2_ref,
               Wc1T_ref, bc1_ref, wc2_ref, bc2_ref, efx_ref):
    ga = ga_ref[...]
    gb = gb_ref[...]
    cd = ga[:, _D:] - gb[:, _D:]
    radial = jnp.sum(cd * cd, axis=1, keepdims=True)
    t1 = (ga[:, :_D] + gb[:, :_D] + radial * wr_ref[...]
          + jnp.dot(ea_ref[...], WeAT_ref[...], preferred_element_type=jnp.float32))
    t1 = _silu(t1)
    ef = jnp.dot(t1, We2T_ref[...], preferred_element_type=jnp.float32) + be2_ref[...]
    ef = _silu(ef)
    cv = jnp.dot(ef, Wc1T_ref[...], preferred_element_type=jnp.float32) + bc1_ref[...]
    cv = _silu(cv)
    c = jnp.dot(cv, wc2_ref[...], preferred_element_type=jnp.float32) + bc2_ref[...]
    trans = jnp.clip(cd * c, -100.0, 100.0)
    lane = lax.broadcasted_iota(jnp.int32, trans.shape, 1)
    aux = jnp.where(lane == 3, 1.0, trans)
    efx_ref[:, :_D] = ef
    efx_ref[:, _D:] = aux


def _edge_call(ga, gb, edge_attr, WeAT, wrr, We2T, be2r, Wc1T, bc1r, wc2, bc2r):
    nblk = _E // _BE
    return pl.pallas_call(
        _edge_body,
        grid=(nblk,),
        in_specs=[
            pl.BlockSpec((_BE, _TW), lambda i: (i, 0)),
            pl.BlockSpec((_BE, _TW), lambda i: (i, 0)),
            pl.BlockSpec((_BE, _DE), lambda i: (i, 0)),
            pl.BlockSpec((_DE, _H), lambda i: (0, 0)),
            pl.BlockSpec((1, _H), lambda i: (0, 0)),
            pl.BlockSpec((_H, _H), lambda i: (0, 0)),
            pl.BlockSpec((1, _H), lambda i: (0, 0)),
            pl.BlockSpec((_H, _H), lambda i: (0, 0)),
            pl.BlockSpec((1, _H), lambda i: (0, 0)),
            pl.BlockSpec((_H, 1), lambda i: (0, 0)),
            pl.BlockSpec((1, 1), lambda i: (0, 0)),
        ],
        out_specs=pl.BlockSpec((_BE, _TW), lambda i: (i, 0)),
        out_shape=jax.ShapeDtypeStruct((_E, _TW), jnp.float32),
    )(ga, gb, edge_attr, WeAT, wrr, We2T, be2r, Wc1T, bc1r, wc2, bc2r)


def _scatter_body(efx_hbm, ei_hbm, zeros_hbm, out_hbm, idx_v, data_v, acc_sh,
                  semi, semd):
    cid = lax.axis_index("c")
    sid = lax.axis_index("s")
    wid = cid * _NS + sid
    base = wid * _EPW

    rbase = sid * _NROWS_PER_SUB
    pltpu.sync_copy(zeros_hbm.at[pl.ds(rbase, _NROWS_PER_SUB)],
                    acc_sh.at[pl.ds(rbase, _NROWS_PER_SUB)])
    plsc.subcore_barrier()

    def start(k, b):
        off = base + k * _CH
        pltpu.async_copy(ei_hbm.at[0, pl.ds(off, _CH)], idx_v.at[b],
                         semi.at[b])
        pltpu.async_copy(efx_hbm.at[pl.ds(off, _CH)], data_v.at[b],
                         semd.at[b])

    def finish(k, b):
        off = base + k * _CH
        pltpu.make_async_copy(ei_hbm.at[0, pl.ds(off, _CH)], idx_v.at[b],
                              semi.at[b]).wait()
        pltpu.make_async_copy(efx_hbm.at[pl.ds(off, _CH)], data_v.at[b],
                              semd.at[b]).wait()
        pltpu.sync_copy(data_v.at[b], acc_sh.at[idx_v.at[b]], add=True)

    start(0, 0)

    def body(k, carry):
        b = lax.rem(k, 2)
        @pl.when(k + 1 < _NCHUNK)
        def _():
            start(k + 1, 1 - b)
        finish(k, b)
        return carry

    lax.fori_loop(0, _NCHUNK, body, 0)

    plsc.subcore_barrier()
    pltpu.sync_copy(acc_sh.at[pl.ds(rbase, _NROWS_PER_SUB)],
                    out_hbm.at[cid, pl.ds(rbase, _NROWS_PER_SUB)])


def _scatter_call(efx, ei, zeros_nt):
    mesh = plsc.VectorSubcoreMesh(core_axis_name="c", subcore_axis_name="s",
                                  num_cores=_NC, num_subcores=_NS)
    f = functools.partial(
        pl.kernel,
        out_type=jax.ShapeDtypeStruct((_NC, _N, _TW), jnp.float32),
        mesh=mesh,
        compiler_params=pltpu.CompilerParams(use_tc_tiling_on_sc=False),
        scratch_types=[
            pltpu.VMEM((2, _CH), jnp.int32),
            pltpu.VMEM((2, _CH, _TW), jnp.float32),
            pltpu.VMEM_SHARED((_N, _TW), jnp.float32),
            pltpu.SemaphoreType.DMA((2,)),
            pltpu.SemaphoreType.DMA((2,)),
        ],
    )(_scatter_body)
    return f(efx, ei, zeros_nt)


def _node_body(h_ref, acc_ref, coord_ref, vel_ref, Wn1hT_ref, Wn1aT_ref,
               bn1_ref, Wn2T_ref, bn2_ref, h2_ref, coord2_ref, vel2_ref):
    acc = acc_ref[0] + acc_ref[1]
    agg = acc[:, :_D]
    trans_sum = acc[:, _D:_D + 3]
    cnt = jnp.clip(acc[:, _D + 3:_D + 4], 1.0, None)
    a_like = trans_sum / cnt
    vel2 = vel_ref[...] + a_like * _STEP
    vel2_ref[...] = vel2
    coord2_ref[...] = coord_ref[...] + vel2 * _STEP
    h = h_ref[...]
    n1 = (jnp.dot(h, Wn1hT_ref[...], preferred_element_type=jnp.float32)
          + jnp.dot(agg, Wn1aT_ref[...], preferred_element_type=jnp.float32)
          + bn1_ref[...])
    n1 = _silu(n1)
    h2_ref[...] = h + jnp.dot(n1, Wn2T_ref[...], preferred_element_type=jnp.float32) + bn2_ref[...]


def _node_call(h, acc, coord, vel, Wn1hT, Wn1aT, bn1r, Wn2T, bn2r):
    nblk = pl.cdiv(_N, _BN)
    return pl.pallas_call(
        _node_body,
        grid=(nblk,),
        in_specs=[
            pl.BlockSpec((_BN, _D), lambda i: (i, 0)),
            pl.BlockSpec((_NC, _BN, _TW), lambda i: (0, i, 0)),
            pl.BlockSpec((_BN, 3), lambda i: (i, 0)),
            pl.BlockSpec((_BN, 3), lambda i: (i, 0)),
            pl.BlockSpec((_D, _H), lambda i: (0, 0)),
            pl.BlockSpec((_H, _H), lambda i: (0, 0)),
            pl.BlockSpec((1, _H), lambda i: (0, 0)),
            pl.BlockSpec((_H, _D), lambda i: (0, 0)),
            pl.BlockSpec((1, _D), lambda i: (0, 0)),
        ],
        out_specs=[
            pl.BlockSpec((_BN, _D), lambda i: (i, 0)),
            pl.BlockSpec((_BN, 3), lambda i: (i, 0)),
            pl.BlockSpec((_BN, 3), lambda i: (i, 0)),
        ],
        out_shape=[
            jax.ShapeDtypeStruct((_N, _D), jnp.float32),
            jax.ShapeDtypeStruct((_N, 3), jnp.float32),
            jax.ShapeDtypeStruct((_N, 3), jnp.float32),
        ],
    )(h, acc, coord, vel, Wn1hT, Wn1aT, bn1r, Wn2T, bn2r)


@jax.jit
def kernel(h, edge_index, coord, vel, edge_attr, We1, be1, We2, be2,
           Wn1, bn1, Wn2, bn2, Wc1, bc1, Wc2, bc2):
    coordp = jnp.pad(coord, ((0, 0), (0, _TW - _D - 3)))
    WaT = We1[:, :_D].T
    WbT = We1[:, _D:2 * _D].T
    wrr = We1[:, 2 * _D].reshape(1, _H)
    WeAT = We1[:, 2 * _D + 1:].T
    be1r = be1.reshape(1, _H)
    be2r = be2.reshape(1, _H)
    bc1r = bc1.reshape(1, _H)
    wc2 = Wc2.T
    bc2r = bc2.reshape(1, 1)
    Wn1hT = Wn1[:, :_D].T
    Wn1aT = Wn1[:, _D:].T
    bn1r = bn1.reshape(1, _H)
    Wn2T = Wn2.T
    bn2r = bn2.reshape(1, _D)
    zeros_nt = jnp.zeros((_N, _TW), jnp.float32)

    ta, tb = _prep_call(h, coordp, WaT, WbT, be1r)
    ga, gb = _gather_call(edge_index, ta, tb)
    efx = _edge_call(ga, gb, edge_attr, WeAT, wrr, We2T := We2.T, be2r,
                     Wc1.T, bc1r, wc2, bc2r)
    acc = _scatter_call(efx, edge_index, zeros_nt)
    h2, coord2, vel2 = _node_call(h, acc, coord, vel, Wn1hT, Wn1aT, bn1r,
                                  Wn2T, bn2r)
    return (h2, coord2, vel2)


# speedup vs baseline: 2.9164x; 1.0967x over previous
import functools
import jax
import jax.numpy as jnp
from jax import lax
from jax.experimental import pallas as pl
from jax.experimental.pallas import tpu as pltpu
from jax.experimental.pallas import tpu_sc as plsc

_N = 10000
_E = 320000
_D = 128
_H = 128
_DE = 16
_STEP = 1.0 / 4.0
_TW = 144
_NC = 2
_NS = 16
_NW = _NC * _NS
_EPW = _E // _NW
_CH = 80
_NCHUNK = _EPW // _CH
_BN = 512
_BE = 1280
_NROWS_PER_SUB = _N // _NS


def _silu(x):
    return x * jax.nn.sigmoid(x)


def _prep_body(h_ref, cp_ref, WaT_ref, WbT_ref, be1_ref, ta_ref, tb_ref):
    h = h_ref[...]
    ta_ref[:, :_D] = jnp.dot(h, WaT_ref[...], preferred_element_type=jnp.float32) + be1_ref[...]
    ta_ref[:, _D:] = cp_ref[...]
    tb_ref[:, :_D] = jnp.dot(h, WbT_ref[...], preferred_element_type=jnp.float32)
    tb_ref[:, _D:] = cp_ref[...]


def _prep_call(h, coordp, WaT, WbT, be1r):
    nblk = pl.cdiv(_N, _BN)
    return pl.pallas_call(
        _prep_body,
        grid=(nblk,),
        in_specs=[
            pl.BlockSpec((_BN, _D), lambda i: (i, 0)),
            pl.BlockSpec((_BN, _TW - _D), lambda i: (i, 0)),
            pl.BlockSpec((_D, _D), lambda i: (0, 0)),
            pl.BlockSpec((_D, _D), lambda i: (0, 0)),
            pl.BlockSpec((1, _D), lambda i: (0, 0)),
        ],
        out_specs=[
            pl.BlockSpec((_BN, _TW), lambda i: (i, 0)),
            pl.BlockSpec((_BN, _TW), lambda i: (i, 0)),
        ],
        out_shape=[
            jax.ShapeDtypeStruct((_N, _TW), jnp.float32),
            jax.ShapeDtypeStruct((_N, _TW), jnp.float32),
        ],
    )(h, coordp, WaT, WbT, be1r)


def _gather_body(ei_hbm, ta_hbm, tb_hbm, ga_hbm, gb_hbm,
                 idxr_v, idxc_v, bufa_v, bufb_v, sema, semb):
    cid = lax.axis_index("c")
    sid = lax.axis_index("s")
    wid = cid * _NS + sid
    base = wid * _EPW

    pltpu.sync_copy(ei_hbm.at[0, pl.ds(base, _EPW)], idxr_v)
    pltpu.sync_copy(ei_hbm.at[1, pl.ds(base, _EPW)], idxc_v)

    def start(k, b):
        o = k * _CH
        pltpu.async_copy(ta_hbm.at[idxr_v.at[pl.ds(o, _CH)]], bufa_v.at[b],
                         sema.at[b])
        pltpu.async_copy(tb_hbm.at[idxc_v.at[pl.ds(o, _CH)]], bufb_v.at[b],
                         semb.at[b])

    def finish(k, b):
        o = k * _CH
        pltpu.make_async_copy(ta_hbm.at[idxr_v.at[pl.ds(o, _CH)]],
                              bufa_v.at[b], sema.at[b]).wait()
        pltpu.make_async_copy(tb_hbm.at[idxc_v.at[pl.ds(o, _CH)]],
                              bufb_v.at[b], semb.at[b]).wait()
        pltpu.sync_copy(bufa_v.at[b], ga_hbm.at[pl.ds(base + o, _CH)])
        pltpu.sync_copy(bufb_v.at[b], gb_hbm.at[pl.ds(base + o, _CH)])

    start(0, 0)

    def body(k, carry):
        b = lax.rem(k, 2)
        @pl.when(k + 1 < _NCHUNK)
        def _():
            start(k + 1, 1 - b)
        finish(k, b)
        return carry

    lax.fori_loop(0, _NCHUNK, body, 0)


def _gather_call(ei, ta, tb):
    mesh = plsc.VectorSubcoreMesh(core_axis_name="c", subcore_axis_name="s",
                                  num_cores=_NC, num_subcores=_NS)
    f = functools.partial(
        pl.kernel,
        out_type=[
            jax.ShapeDtypeStruct((_E, _TW), jnp.float32),
            jax.ShapeDtypeStruct((_E, _TW), jnp.float32),
        ],
        mesh=mesh,
        compiler_params=pltpu.CompilerParams(use_tc_tiling_on_sc=False),
        scratch_types=[
            pltpu.VMEM((_EPW,), jnp.int32),
            pltpu.VMEM((_EPW,), jnp.int32),
            pltpu.VMEM((2, _CH, _TW), jnp.float32),
            pltpu.VMEM((2, _CH, _TW), jnp.float32),
            pltpu.SemaphoreType.DMA((2,)),
            pltpu.SemaphoreType.DMA((2,)),
        ],
    )(_gather_body)
    return f(ei, ta, tb)


def _edge_body(ga_ref, gb_ref, ea_ref, WeAT_ref, wr_ref, We2T_ref, be2_ref,
               Wc1T_ref, bc1_ref, wc2_ref, bc2_ref, efx_ref):
    ga = ga_ref[...]
    gb = gb_ref[...]
    cd = ga[:, _D:] - gb[:, _D:]
    radial = jnp.sum(cd * cd, axis=1, keepdims=True)
    t1 = (ga[:, :_D] + gb[:, :_D] + radial * wr_ref[...]
          + jnp.dot(ea_ref[...], WeAT_ref[...], preferred_element_type=jnp.float32))
    t1 = _silu(t1)
    ef = jnp.dot(t1, We2T_ref[...], preferred_element_type=jnp.float32) + be2_ref[...]
    ef = _silu(ef)
    cv = jnp.dot(ef, Wc1T_ref[...], preferred_element_type=jnp.float32) + bc1_ref[...]
    cv = _silu(cv)
    c = jnp.dot(cv, wc2_ref[...], preferred_element_type=jnp.float32) + bc2_ref[...]
    trans = jnp.clip(cd * c, -100.0, 100.0)
    lane = lax.broadcasted_iota(jnp.int32, trans.shape, 1)
    aux = jnp.where(lane == 3, 1.0, trans)
    efx_ref[:, :_D] = ef
    efx_ref[:, _D:] = aux


def _edge_call(ga, gb, edge_attr, WeAT, wrr, We2T, be2r, Wc1T, bc1r, wc2, bc2r):
    nblk = _E // _BE
    return pl.pallas_call(
        _edge_body,
        grid=(nblk,),
        in_specs=[
            pl.BlockSpec((_BE, _TW), lambda i: (i, 0)),
            pl.BlockSpec((_BE, _TW), lambda i: (i, 0)),
            pl.BlockSpec((_BE, _DE), lambda i: (i, 0)),
            pl.BlockSpec((_DE, _H), lambda i: (0, 0)),
            pl.BlockSpec((1, _H), lambda i: (0, 0)),
            pl.BlockSpec((_H, _H), lambda i: (0, 0)),
            pl.BlockSpec((1, _H), lambda i: (0, 0)),
            pl.BlockSpec((_H, _H), lambda i: (0, 0)),
            pl.BlockSpec((1, _H), lambda i: (0, 0)),
            pl.BlockSpec((_H, 1), lambda i: (0, 0)),
            pl.BlockSpec((1, 1), lambda i: (0, 0)),
        ],
        out_specs=pl.BlockSpec((_BE, _TW), lambda i: (i, 0)),
        out_shape=jax.ShapeDtypeStruct((_E, _TW), jnp.float32),
    )(ga, gb, edge_attr, WeAT, wrr, We2T, be2r, Wc1T, bc1r, wc2, bc2r)


def _scatter_body(efx_hbm, ei_hbm, zeros_hbm, out_hbm, idx_v, data_v, acc_sh,
                  semi, semd):
    cid = lax.axis_index("c")
    sid = lax.axis_index("s")
    wid = cid * _NS + sid
    base = wid * _EPW

    rbase = sid * _NROWS_PER_SUB
    pltpu.sync_copy(zeros_hbm.at[pl.ds(rbase, _NROWS_PER_SUB)],
                    acc_sh.at[pl.ds(rbase, _NROWS_PER_SUB)])
    plsc.subcore_barrier()

    def start(k, b):
        off = base + k * _CH
        pltpu.async_copy(ei_hbm.at[0, pl.ds(off, _CH)], idx_v.at[b],
                         semi.at[b])
        pltpu.async_copy(efx_hbm.at[pl.ds(off, _CH)], data_v.at[b],
                         semd.at[b])

    def finish(k, b):
        off = base + k * _CH
        pltpu.make_async_copy(ei_hbm.at[0, pl.ds(off, _CH)], idx_v.at[b],
                              semi.at[b]).wait()
        pltpu.make_async_copy(efx_hbm.at[pl.ds(off, _CH)], data_v.at[b],
                              semd.at[b]).wait()
        pltpu.sync_copy(data_v.at[b], acc_sh.at[idx_v.at[b]], add=True)

    start(0, 0)

    def body(k, carry):
        b = lax.rem(k, 2)
        @pl.when(k + 1 < _NCHUNK)
        def _():
            start(k + 1, 1 - b)
        finish(k, b)
        return carry

    lax.fori_loop(0, _NCHUNK, body, 0)

    plsc.subcore_barrier()
    pltpu.sync_copy(acc_sh.at[pl.ds(rbase, _NROWS_PER_SUB)],
                    out_hbm.at[cid, pl.ds(rbase, _NROWS_PER_SUB)])


def _scatter_call(efx, ei, zeros_nt):
    mesh = plsc.VectorSubcoreMesh(core_axis_name="c", subcore_axis_name="s",
                                  num_cores=_NC, num_subcores=_NS)
    f = functools.partial(
        pl.kernel,
        out_type=jax.ShapeDtypeStruct((_NC, _N, _TW), jnp.float32),
        mesh=mesh,
        compiler_params=pltpu.CompilerParams(use_tc_tiling_on_sc=False),
        scratch_types=[
            pltpu.VMEM((2, _CH), jnp.int32),
            pltpu.VMEM((2, _CH, _TW), jnp.float32),
            pltpu.VMEM_SHARED((_N, _TW), jnp.float32),
            pltpu.SemaphoreType.DMA((2,)),
            pltpu.SemaphoreType.DMA((2,)),
        ],
    )(_scatter_body)
    return f(efx, ei, zeros_nt)


def _node_body(h_ref, acc_ref, coord_ref, vel_ref, Wn1hT_ref, Wn1aT_ref,
               bn1_ref, Wn2T_ref, bn2_ref, h2_ref, coord2_ref, vel2_ref):
    acc = acc_ref[0] + acc_ref[1]
    agg = acc[:, :_D]
    trans_sum = acc[:, _D:_D + 3]
    cnt = jnp.clip(acc[:, _D + 3:_D + 4], 1.0, None)
    a_like = trans_sum / cnt
    vel2 = vel_ref[...] + a_like * _STEP
    vel2_ref[...] = vel2
    coord2_ref[...] = coord_ref[...] + vel2 * _STEP
    h = h_ref[...]
    n1 = (jnp.dot(h, Wn1hT_ref[...], preferred_element_type=jnp.float32)
          + jnp.dot(agg, Wn1aT_ref[...], preferred_element_type=jnp.float32)
          + bn1_ref[...])
    n1 = _silu(n1)
    h2_ref[...] = h + jnp.dot(n1, Wn2T_ref[...], preferred_element_type=jnp.float32) + bn2_ref[...]


def _node_call(h, acc, coord, vel, Wn1hT, Wn1aT, bn1r, Wn2T, bn2r):
    nblk = pl.cdiv(_N, _BN)
    return pl.pallas_call(
        _node_body,
        grid=(nblk,),
        in_specs=[
            pl.BlockSpec((_BN, _D), lambda i: (i, 0)),
            pl.BlockSpec((_NC, _BN, _TW), lambda i: (0, i, 0)),
            pl.BlockSpec((_BN, 3), lambda i: (i, 0)),
            pl.BlockSpec((_BN, 3), lambda i: (i, 0)),
            pl.BlockSpec((_D, _H), lambda i: (0, 0)),
            pl.BlockSpec((_H, _H), lambda i: (0, 0)),
            pl.BlockSpec((1, _H), lambda i: (0, 0)),
            pl.BlockSpec((_H, _D), lambda i: (0, 0)),
            pl.BlockSpec((1, _D), lambda i: (0, 0)),
        ],
        out_specs=[
            pl.BlockSpec((_BN, _D), lambda i: (i, 0)),
            pl.BlockSpec((_BN, 3), lambda i: (i, 0)),
            pl.BlockSpec((_BN, 3), lambda i: (i, 0)),
        ],
        out_shape=[
            jax.ShapeDtypeStruct((_N, _D), jnp.float32),
            jax.ShapeDtypeStruct((_N, 3), jnp.float32),
            jax.ShapeDtypeStruct((_N, 3), jnp.float32),
        ],
    )(h, acc, coord, vel, Wn1hT, Wn1aT, bn1r, Wn2T, bn2r)


@jax.jit
def kernel(h, edge_index, coord, vel, edge_attr, We1, be1, We2, be2,
           Wn1, bn1, Wn2, bn2, Wc1, bc1, Wc2, bc2):
    coordp = jnp.pad(coord, ((0, 0), (0, _TW - _D - 3)))
    WaT = We1[:, :_D].T
    WbT = We1[:, _D:2 * _D].T
    wrr = We1[:, 2 * _D].reshape(1, _H)
    WeAT = We1[:, 2 * _D + 1:].T
    be1r = be1.reshape(1, _H)
    be2r = be2.reshape(1, _H)
    bc1r = bc1.reshape(1, _H)
    wc2 = Wc2.T
    bc2r = bc2.reshape(1, 1)
    Wn1hT = Wn1[:, :_D].T
    Wn1aT = Wn1[:, _D:].T
    bn1r = bn1.reshape(1, _H)
    Wn2T = Wn2.T
    bn2r = bn2.reshape(1, _D)
    zeros_nt = jnp.zeros((_N, _TW), jnp.float32)

    ta, tb = _prep_call(h, coordp, WaT, WbT, be1r)
    ga, gb = _gather_call(edge_index, ta, tb)
    efx = _edge_call(ga, gb, edge_attr, WeAT, wrr, We2T := We2.T, be2r,
                     Wc1.T, bc1r, wc2, bc2r)
    acc = _scatter_call(efx, edge_index, zeros_nt)
    h2, coord2, vel2 = _node_call(h, acc, coord, vel, Wn1hT, Wn1aT, bn1r,
                                  Wn2T, bn2r)
    return (h2, coord2, vel2)


# speedup vs baseline: 3.0430x; 1.0434x over previous
import functools
import jax
import jax.numpy as jnp
from jax import lax
from jax.experimental import pallas as pl
from jax.experimental.pallas import tpu as pltpu
from jax.experimental.pallas import tpu_sc as plsc

_N = 10000
_E = 320000
_D = 128
_H = 128
_DE = 16
_STEP = 1.0 / 4.0
_TW = 144
_NC = 2
_NS = 16
_NW = _NC * _NS
_EPW = _E // _NW
_CH = 80
_NCHUNK = _EPW // _CH
_BN = 512
_BE = 2560
_NROWS_PER_SUB = _N // _NS


def _silu(x):
    return x * jax.nn.sigmoid(x)


def _prep_body(h_ref, cp_ref, WaT_ref, WbT_ref, be1_ref, ta_ref, tb_ref):
    h = h_ref[...]
    ta_ref[:, :_D] = jnp.dot(h, WaT_ref[...], preferred_element_type=jnp.float32) + be1_ref[...]
    ta_ref[:, _D:] = cp_ref[...]
    tb_ref[:, :_D] = jnp.dot(h, WbT_ref[...], preferred_element_type=jnp.float32)
    tb_ref[:, _D:] = cp_ref[...]


def _prep_call(h, coordp, WaT, WbT, be1r):
    nblk = pl.cdiv(_N, _BN)
    return pl.pallas_call(
        _prep_body,
        grid=(nblk,),
        in_specs=[
            pl.BlockSpec((_BN, _D), lambda i: (i, 0)),
            pl.BlockSpec((_BN, _TW - _D), lambda i: (i, 0)),
            pl.BlockSpec((_D, _D), lambda i: (0, 0)),
            pl.BlockSpec((_D, _D), lambda i: (0, 0)),
            pl.BlockSpec((1, _D), lambda i: (0, 0)),
        ],
        out_specs=[
            pl.BlockSpec((_BN, _TW), lambda i: (i, 0)),
            pl.BlockSpec((_BN, _TW), lambda i: (i, 0)),
        ],
        out_shape=[
            jax.ShapeDtypeStruct((_N, _TW), jnp.float32),
            jax.ShapeDtypeStruct((_N, _TW), jnp.float32),
        ],
    )(h, coordp, WaT, WbT, be1r)


def _gather_body(ei_hbm, ta_hbm, tb_hbm, ga_hbm, gb_hbm,
                 idxr_v, idxc_v, bufa_v, bufb_v, sema, semb):
    cid = lax.axis_index("c")
    sid = lax.axis_index("s")
    wid = cid * _NS + sid
    base = wid * _EPW

    pltpu.sync_copy(ei_hbm.at[0, pl.ds(base, _EPW)], idxr_v)
    pltpu.sync_copy(ei_hbm.at[1, pl.ds(base, _EPW)], idxc_v)

    def start(k, b):
        o = k * _CH
        pltpu.async_copy(ta_hbm.at[idxr_v.at[pl.ds(o, _CH)]], bufa_v.at[b],
                         sema.at[b])
        pltpu.async_copy(tb_hbm.at[idxc_v.at[pl.ds(o, _CH)]], bufb_v.at[b],
                         semb.at[b])

    def finish(k, b):
        o = k * _CH
        pltpu.make_async_copy(ta_hbm.at[idxr_v.at[pl.ds(o, _CH)]],
                              bufa_v.at[b], sema.at[b]).wait()
        pltpu.make_async_copy(tb_hbm.at[idxc_v.at[pl.ds(o, _CH)]],
                              bufb_v.at[b], semb.at[b]).wait()
        pltpu.sync_copy(bufa_v.at[b], ga_hbm.at[pl.ds(base + o, _CH)])
        pltpu.sync_copy(bufb_v.at[b], gb_hbm.at[pl.ds(base + o, _CH)])

    start(0, 0)

    def body(k, carry):
        b = lax.rem(k, 2)
        @pl.when(k + 1 < _NCHUNK)
        def _():
            start(k + 1, 1 - b)
        finish(k, b)
        return carry

    lax.fori_loop(0, _NCHUNK, body, 0)


def _gather_call(ei, ta, tb):
    mesh = plsc.VectorSubcoreMesh(core_axis_name="c", subcore_axis_name="s",
                                  num_cores=_NC, num_subcores=_NS)
    f = functools.partial(
        pl.kernel,
        out_type=[
            jax.ShapeDtypeStruct((_E, _TW), jnp.float32),
            jax.ShapeDtypeStruct((_E, _TW), jnp.float32),
        ],
        mesh=mesh,
        compiler_params=pltpu.CompilerParams(use_tc_tiling_on_sc=False),
        scratch_types=[
            pltpu.VMEM((_EPW,), jnp.int32),
            pltpu.VMEM((_EPW,), jnp.int32),
            pltpu.VMEM((2, _CH, _TW), jnp.float32),
            pltpu.VMEM((2, _CH, _TW), jnp.float32),
            pltpu.SemaphoreType.DMA((2,)),
            pltpu.SemaphoreType.DMA((2,)),
        ],
    )(_gather_body)
    return f(ei, ta, tb)


def _edge_body(ga_ref, gb_ref, ea_ref, WeAT_ref, wr_ref, We2T_ref, be2_ref,
               Wc1T_ref, bc1_ref, wc2_ref, bc2_ref, efx_ref):
    ga = ga_ref[...]
    gb = gb_ref[...]
    cd = ga[:, _D:] - gb[:, _D:]
    radial = jnp.sum(cd * cd, axis=1, keepdims=True)
    t1 = (ga[:, :_D] + gb[:, :_D] + radial * wr_ref[...]
          + jnp.dot(ea_ref[...], WeAT_ref[...], preferred_element_type=jnp.float32))
    t1 = _silu(t1)
    ef = jnp.dot(t1, We2T_ref[...], preferred_element_type=jnp.float32) + be2_ref[...]
    ef = _silu(ef)
    cv = jnp.dot(ef, Wc1T_ref[...], preferred_element_type=jnp.float32) + bc1_ref[...]
    cv = _silu(cv)
    c = jnp.dot(cv, wc2_ref[...], preferred_element_type=jnp.float32) + bc2_ref[...]
    trans = jnp.clip(cd * c, -100.0, 100.0)
    lane = lax.broadcasted_iota(jnp.int32, trans.shape, 1)
    aux = jnp.where(lane == 3, 1.0, trans)
    efx_ref[:, :_D] = ef
    efx_ref[:, _D:] = aux


def _edge_call(ga, gb, edge_attr, WeAT, wrr, We2T, be2r, Wc1T, bc1r, wc2, bc2r):
    nblk = _E // _BE
    return pl.pallas_call(
        _edge_body,
        grid=(nblk,),
        in_specs=[
            pl.BlockSpec((_BE, _TW), lambda i: (i, 0)),
            pl.BlockSpec((_BE, _TW), lambda i: (i, 0)),
            pl.BlockSpec((_BE, _DE), lambda i: (i, 0)),
            pl.BlockSpec((_DE, _H), lambda i: (0, 0)),
            pl.BlockSpec((1, _H), lambda i: (0, 0)),
            pl.BlockSpec((_H, _H), lambda i: (0, 0)),
            pl.BlockSpec((1, _H), lambda i: (0, 0)),
            pl.BlockSpec((_H, _H), lambda i: (0, 0)),
            pl.BlockSpec((1, _H), lambda i: (0, 0)),
            pl.BlockSpec((_H, 1), lambda i: (0, 0)),
            pl.BlockSpec((1, 1), lambda i: (0, 0)),
        ],
        out_specs=pl.BlockSpec((_BE, _TW), lambda i: (i, 0)),
        out_shape=jax.ShapeDtypeStruct((_E, _TW), jnp.float32),
    )(ga, gb, edge_attr, WeAT, wrr, We2T, be2r, Wc1T, bc1r, wc2, bc2r)


def _scatter_body(efx_hbm, ei_hbm, zeros_hbm, out_hbm, idx_v, data_v, acc_sh,
                  semi, semd):
    cid = lax.axis_index("c")
    sid = lax.axis_index("s")
    wid = cid * _NS + sid
    base = wid * _EPW

    rbase = sid * _NROWS_PER_SUB
    pltpu.sync_copy(zeros_hbm.at[pl.ds(rbase, _NROWS_PER_SUB)],
                    acc_sh.at[pl.ds(rbase, _NROWS_PER_SUB)])
    plsc.subcore_barrier()

    def start(k, b):
        off = base + k * _CH
        pltpu.async_copy(ei_hbm.at[0, pl.ds(off, _CH)], idx_v.at[b],
                         semi.at[b])
        pltpu.async_copy(efx_hbm.at[pl.ds(off, _CH)], data_v.at[b],
                         semd.at[b])

    def finish(k, b):
        off = base + k * _CH
        pltpu.make_async_copy(ei_hbm.at[0, pl.ds(off, _CH)], idx_v.at[b],
                              semi.at[b]).wait()
        pltpu.make_async_copy(efx_hbm.at[pl.ds(off, _CH)], data_v.at[b],
                              semd.at[b]).wait()
        pltpu.sync_copy(data_v.at[b], acc_sh.at[idx_v.at[b]], add=True)

    start(0, 0)

    def body(k, carry):
        b = lax.rem(k, 2)
        @pl.when(k + 1 < _NCHUNK)
        def _():
            start(k + 1, 1 - b)
        finish(k, b)
        return carry

    lax.fori_loop(0, _NCHUNK, body, 0)

    plsc.subcore_barrier()
    pltpu.sync_copy(acc_sh.at[pl.ds(rbase, _NROWS_PER_SUB)],
                    out_hbm.at[cid, pl.ds(rbase, _NROWS_PER_SUB)])


def _scatter_call(efx, ei, zeros_nt):
    mesh = plsc.VectorSubcoreMesh(core_axis_name="c", subcore_axis_name="s",
                                  num_cores=_NC, num_subcores=_NS)
    f = functools.partial(
        pl.kernel,
        out_type=jax.ShapeDtypeStruct((_NC, _N, _TW), jnp.float32),
        mesh=mesh,
        compiler_params=pltpu.CompilerParams(use_tc_tiling_on_sc=False),
        scratch_types=[
            pltpu.VMEM((2, _CH), jnp.int32),
            pltpu.VMEM((2, _CH, _TW), jnp.float32),
            pltpu.VMEM_SHARED((_N, _TW), jnp.float32),
            pltpu.SemaphoreType.DMA((2,)),
            pltpu.SemaphoreType.DMA((2,)),
        ],
    )(_scatter_body)
    return f(efx, ei, zeros_nt)


def _node_body(h_ref, acc_ref, coord_ref, vel_ref, Wn1hT_ref, Wn1aT_ref,
               bn1_ref, Wn2T_ref, bn2_ref, h2_ref, coord2_ref, vel2_ref):
    acc = acc_ref[0] + acc_ref[1]
    agg = acc[:, :_D]
    trans_sum = acc[:, _D:_D + 3]
    cnt = jnp.clip(acc[:, _D + 3:_D + 4], 1.0, None)
    a_like = trans_sum / cnt
    vel2 = vel_ref[...] + a_like * _STEP
    vel2_ref[...] = vel2
    coord2_ref[...] = coord_ref[...] + vel2 * _STEP
    h = h_ref[...]
    n1 = (jnp.dot(h, Wn1hT_ref[...], preferred_element_type=jnp.float32)
          + jnp.dot(agg, Wn1aT_ref[...], preferred_element_type=jnp.float32)
          + bn1_ref[...])
    n1 = _silu(n1)
    h2_ref[...] = h + jnp.dot(n1, Wn2T_ref[...], preferred_element_type=jnp.float32) + bn2_ref[...]


def _node_call(h, acc, coord, vel, Wn1hT, Wn1aT, bn1r, Wn2T, bn2r):
    nblk = pl.cdiv(_N, _BN)
    return pl.pallas_call(
        _node_body,
        grid=(nblk,),
        in_specs=[
            pl.BlockSpec((_BN, _D), lambda i: (i, 0)),
            pl.BlockSpec((_NC, _BN, _TW), lambda i: (0, i, 0)),
            pl.BlockSpec((_BN, 3), lambda i: (i, 0)),
            pl.BlockSpec((_BN, 3), lambda i: (i, 0)),
            pl.BlockSpec((_D, _H), lambda i: (0, 0)),
            pl.BlockSpec((_H, _H), lambda i: (0, 0)),
            pl.BlockSpec((1, _H), lambda i: (0, 0)),
            pl.BlockSpec((_H, _D), lambda i: (0, 0)),
            pl.BlockSpec((1, _D), lambda i: (0, 0)),
        ],
        out_specs=[
            pl.BlockSpec((_BN, _D), lambda i: (i, 0)),
            pl.BlockSpec((_BN, 3), lambda i: (i, 0)),
            pl.BlockSpec((_BN, 3), lambda i: (i, 0)),
        ],
        out_shape=[
            jax.ShapeDtypeStruct((_N, _D), jnp.float32),
            jax.ShapeDtypeStruct((_N, 3), jnp.float32),
            jax.ShapeDtypeStruct((_N, 3), jnp.float32),
        ],
    )(h, acc, coord, vel, Wn1hT, Wn1aT, bn1r, Wn2T, bn2r)


@jax.jit
def kernel(h, edge_index, coord, vel, edge_attr, We1, be1, We2, be2,
           Wn1, bn1, Wn2, bn2, Wc1, bc1, Wc2, bc2):
    coordp = jnp.pad(coord, ((0, 0), (0, _TW - _D - 3)))
    WaT = We1[:, :_D].T
    WbT = We1[:, _D:2 * _D].T
    wrr = We1[:, 2 * _D].reshape(1, _H)
    WeAT = We1[:, 2 * _D + 1:].T
    be1r = be1.reshape(1, _H)
    be2r = be2.reshape(1, _H)
    bc1r = bc1.reshape(1, _H)
    wc2 = Wc2.T
    bc2r = bc2.reshape(1, 1)
    Wn1hT = Wn1[:, :_D].T
    Wn1aT = Wn1[:, _D:].T
    bn1r = bn1.reshape(1, _H)
    Wn2T = Wn2.T
    bn2r = bn2.reshape(1, _D)
    zeros_nt = jnp.zeros((_N, _TW), jnp.float32)

    ta, tb = _prep_call(h, coordp, WaT, WbT, be1r)
    ga, gb = _gather_call(edge_index, ta, tb)
    efx = _edge_call(ga, gb, edge_attr, WeAT, wrr, We2T := We2.T, be2r,
                     Wc1.T, bc1r, wc2, bc2r)
    acc = _scatter_call(efx, edge_index, zeros_nt)
    h2, coord2, vel2 = _node_call(h, acc, coord, vel, Wn1hT, Wn1aT, bn1r,
                                  Wn2T, bn2r)
    return (h2, coord2, vel2)
